# baseline (speedup 1.0000x reference)
import jax
import jax.numpy as jnp
from jax.experimental import pallas as pl

_ROWS_PER_BLOCK = 512


def _combine_body(x_ref, w_ref, o_ref):
    x = x_ref[...]
    w = w_ref[...]
    D = x.shape[-1]
    ones = jnp.ones((1, D), dtype=x.dtype)
    w0 = jax.lax.dot_general(w[:, 0:1], ones, (((1,), (0,)), ((), ())),
                             preferred_element_type=x.dtype)
    w1 = jax.lax.dot_general(w[:, 1:2], ones, (((1,), (0,)), ((), ())),
                             preferred_element_type=x.dtype)
    o_ref[...] = x[:, 0, :] * w0 + x[:, 1, :] * w1


def kernel(combined_output, weights):
    B, T, K, D = combined_output.shape
    N = B * T
    x = combined_output.reshape(N, K, D)
    w = weights.reshape(N, K)
    R = _ROWS_PER_BLOCK
    grid = (N // R,)
    out = pl.pallas_call(
        _combine_body,
        grid=grid,
        in_specs=[
            pl.BlockSpec((R, K, D), lambda i: (i, 0, 0)),
            pl.BlockSpec((R, K), lambda i: (i, 0)),
        ],
        out_specs=pl.BlockSpec((R, D), lambda i: (i, 0)),
        out_shape=jax.ShapeDtypeStruct((N, D), combined_output.dtype),
    )(x, w)
    return out.reshape(B, T, D)
